# baseline (speedup 1.0000x reference)
import jax
import jax.numpy as jnp
from jax.experimental import pallas as pl

_B = 2048
_T = 2
_NB = 1000


def _onehot(ids, nb, b):
    return (ids == jax.lax.broadcasted_iota(jnp.int32, (nb, b), 1)).astype(jnp.float32)


def _pre_kernel(x_ref, b_ref, ws_ref, as_ref, xs_ref, asrc_ref, s0_ref, m_ref):
    i = pl.program_id(0)
    xb = x_ref[...]
    xs = jnp.dot(xb, ws_ref[...], preferred_element_type=jnp.float32)
    xs_ref[...] = xs
    a = jnp.dot(xs, as_ref[...], preferred_element_type=jnp.float32)
    asrc_ref[...] = a
    oh = _onehot(b_ref[...], xb.shape[0], s0_ref.shape[0])
    part = jax.lax.dot_general(oh.astype(jnp.bfloat16), xb.astype(jnp.bfloat16),
                               (((0,), (0,)), ((), ())),
                               preferred_element_type=jnp.float32)
    blkmax = jnp.max(a)

    @pl.when(i == 0)
    def _():
        s0_ref[...] = part
        m_ref[...] = jnp.full((1, 1), blkmax, jnp.float32)

    @pl.when(i != 0)
    def _():
        s0_ref[...] += part
        m_ref[...] = jnp.maximum(m_ref[...], blkmax)


def _adst_kernel(out_ref, wd_ref, ad_ref, ms_ref, adst_ref, m_ref):
    xd = jnp.dot(out_ref[...], wd_ref[...], preferred_element_type=jnp.float32)
    a = jnp.dot(xd, ad_ref[...], preferred_element_type=jnp.float32)
    adst_ref[...] = a
    m_ref[...] = jnp.maximum(ms_ref[...] + jnp.max(a), 0.0)


def _pass_kernel(xs_ref, asrc_ref, b_ref, adst_ref, m_ref, den_ref, num_ref):
    i = pl.program_id(0)
    xs = xs_ref[...]
    oh = _onehot(b_ref[...], xs.shape[0], den_ref.shape[0])
    g = jnp.dot(oh, adst_ref[...], preferred_element_type=jnp.float32)
    alpha = asrc_ref[...] + g
    alpha = jnp.where(alpha > 0, alpha, 0.01 * alpha)
    ex = jnp.exp(alpha - m_ref[0, 0])
    den_part = jax.lax.dot_general(oh, ex, (((0,), (0,)), ((), ())),
                                   preferred_element_type=jnp.float32)
    num_part = jax.lax.dot_general(oh.astype(jnp.bfloat16),
                                   (ex * xs).astype(jnp.bfloat16),
                                   (((0,), (0,)), ((), ())),
                                   preferred_element_type=jnp.float32)

    @pl.when(i == 0)
    def _():
        den_ref[...] = den_part
        num_ref[...] = num_part

    @pl.when(i != 0)
    def _():
        den_ref[...] += den_part
        num_ref[...] += num_part


def _gru_kernel(den_ref, num_ref, bg_ref, out_ref, wih_ref, whh_ref,
                bih_ref, bhh_ref, o_ref):
    d = out_ref.shape[1]
    agg = num_ref[...] / (den_ref[...] + 1e-16) + bg_ref[...]
    h = jnp.where(agg > 0, agg, jnp.exp(jnp.minimum(agg, 0.0)) - 1.0)
    hp = out_ref[...]
    gi = jax.lax.dot_general(h, wih_ref[...], (((1,), (1,)), ((), ())),
                             preferred_element_type=jnp.float32) + bih_ref[...]
    gh = jax.lax.dot_general(hp, whh_ref[...], (((1,), (1,)), ((), ())),
                             preferred_element_type=jnp.float32) + bhh_ref[...]
    r = jax.nn.sigmoid(gi[:, :d] + gh[:, :d])
    z = jax.nn.sigmoid(gi[:, d:2 * d] + gh[:, d:2 * d])
    n = jnp.tanh(gi[:, 2 * d:] + r * gh[:, 2 * d:])
    g = (1.0 - z) * n + z * hp
    o_ref[...] = g * jax.nn.sigmoid(g)


def _lin_kernel(out_ref, wl_ref, bl_ref, o_ref):
    o_ref[...] = jax.lax.dot_general(out_ref[...], wl_ref[...],
                                     (((1,), (1,)), ((), ())),
                                     preferred_element_type=jnp.float32) + bl_ref[...]


def kernel(x, edge_index, batch, W_src, W_dst, att_src, att_dst, bias_gat,
           W_ih, W_hh, b_ih, b_hh, W_lin, b_lin):
    n, d = x.shape
    b = _B
    out_dim = W_lin.shape[0]
    grid = (n // _NB,)
    batch2 = batch.reshape(n, 1)

    xs, a_src, out, m_src = pl.pallas_call(
        _pre_kernel,
        grid=grid,
        in_specs=[
            pl.BlockSpec((_NB, d), lambda i: (i, 0)),
            pl.BlockSpec((_NB, 1), lambda i: (i, 0)),
            pl.BlockSpec((d, d), lambda i: (0, 0)),
            pl.BlockSpec((d, 1), lambda i: (0, 0)),
        ],
        out_specs=[
            pl.BlockSpec((_NB, d), lambda i: (i, 0)),
            pl.BlockSpec((_NB, 1), lambda i: (i, 0)),
            pl.BlockSpec((b, d), lambda i: (0, 0)),
            pl.BlockSpec((1, 1), lambda i: (0, 0)),
        ],
        out_shape=[
            jax.ShapeDtypeStruct((n, d), jnp.float32),
            jax.ShapeDtypeStruct((n, 1), jnp.float32),
            jax.ShapeDtypeStruct((b, d), jnp.float32),
            jax.ShapeDtypeStruct((1, 1), jnp.float32),
        ],
    )(x, batch2, W_src, att_src.reshape(d, 1))

    adst_call = pl.pallas_call(
        _adst_kernel,
        out_shape=[
            jax.ShapeDtypeStruct((b, 1), jnp.float32),
            jax.ShapeDtypeStruct((1, 1), jnp.float32),
        ],
    )

    pass_call = pl.pallas_call(
        _pass_kernel,
        grid=grid,
        in_specs=[
            pl.BlockSpec((_NB, d), lambda i: (i, 0)),
            pl.BlockSpec((_NB, 1), lambda i: (i, 0)),
            pl.BlockSpec((_NB, 1), lambda i: (i, 0)),
            pl.BlockSpec((b, 1), lambda i: (0, 0)),
            pl.BlockSpec((1, 1), lambda i: (0, 0)),
        ],
        out_specs=[
            pl.BlockSpec((b, 1), lambda i: (0, 0)),
            pl.BlockSpec((b, d), lambda i: (0, 0)),
        ],
        out_shape=[
            jax.ShapeDtypeStruct((b, 1), jnp.float32),
            jax.ShapeDtypeStruct((b, d), jnp.float32),
        ],
    )

    gru_call = pl.pallas_call(
        _gru_kernel,
        out_shape=jax.ShapeDtypeStruct((b, d), jnp.float32),
    )

    for _ in range(_T):
        a_dst, m = adst_call(out, W_dst, att_dst.reshape(d, 1), m_src)
        den, num = pass_call(xs, a_src, batch2, a_dst, m)
        out = gru_call(den, num, bias_gat.reshape(1, d), out, W_ih, W_hh,
                       b_ih.reshape(1, 3 * d), b_hh.reshape(1, 3 * d))

    return pl.pallas_call(
        _lin_kernel,
        out_shape=jax.ShapeDtypeStruct((b, out_dim), jnp.float32),
    )(out, W_lin, b_lin.reshape(1, out_dim))


# speedup vs baseline: 1.0086x; 1.0086x over previous
import jax
import jax.numpy as jnp
from jax.experimental import pallas as pl

_B = 2048
_T = 2
_NB = 1000


def _onehot(ids, nb, b):
    return (ids == jax.lax.broadcasted_iota(jnp.int32, (nb, b), 1)).astype(jnp.float32)


def _pre_kernel(x_ref, b_ref, ws_ref, as_ref, xs_ref, asrc_ref, s0_ref, m_ref):
    i = pl.program_id(0)
    xb = x_ref[...]
    xs = jnp.dot(xb, ws_ref[...], preferred_element_type=jnp.float32)
    xs_ref[...] = xs
    a = jnp.dot(xs, as_ref[...], preferred_element_type=jnp.float32)
    asrc_ref[...] = a
    oh = _onehot(b_ref[...], xb.shape[0], s0_ref.shape[0])
    part = jax.lax.dot_general(oh, xb, (((0,), (0,)), ((), ())),
                               preferred_element_type=jnp.float32)
    blkmax = jnp.max(a)

    @pl.when(i == 0)
    def _():
        s0_ref[...] = part
        m_ref[...] = jnp.full((1, 1), blkmax, jnp.float32)

    @pl.when(i != 0)
    def _():
        s0_ref[...] += part
        m_ref[...] = jnp.maximum(m_ref[...], blkmax)


def _adst_kernel(out_ref, wd_ref, ad_ref, ms_ref, adst_ref, m_ref):
    xd = jnp.dot(out_ref[...], wd_ref[...], preferred_element_type=jnp.float32)
    a = jnp.dot(xd, ad_ref[...], preferred_element_type=jnp.float32)
    adst_ref[...] = a
    m_ref[...] = jnp.maximum(ms_ref[...] + jnp.max(a), 0.0)


def _pass_kernel(xs_ref, asrc_ref, b_ref, adst_ref, m_ref, den_ref, num_ref):
    i = pl.program_id(0)
    xs = xs_ref[...]
    oh = _onehot(b_ref[...], xs.shape[0], den_ref.shape[0])
    g = jnp.dot(oh, adst_ref[...], preferred_element_type=jnp.float32)
    alpha = asrc_ref[...] + g
    alpha = jnp.where(alpha > 0, alpha, 0.01 * alpha)
    ex = jnp.exp(alpha - m_ref[0, 0])
    den_part = jax.lax.dot_general(oh, ex, (((0,), (0,)), ((), ())),
                                   preferred_element_type=jnp.float32)
    num_part = jax.lax.dot_general(oh, ex * xs, (((0,), (0,)), ((), ())),
                                   preferred_element_type=jnp.float32)

    @pl.when(i == 0)
    def _():
        den_ref[...] = den_part
        num_ref[...] = num_part

    @pl.when(i != 0)
    def _():
        den_ref[...] += den_part
        num_ref[...] += num_part


def _gru_kernel(den_ref, num_ref, bg_ref, out_ref, wih_ref, whh_ref,
                bih_ref, bhh_ref, o_ref):
    d = out_ref.shape[1]
    agg = num_ref[...] / (den_ref[...] + 1e-16) + bg_ref[...]
    h = jnp.where(agg > 0, agg, jnp.exp(jnp.minimum(agg, 0.0)) - 1.0)
    hp = out_ref[...]
    gi = jax.lax.dot_general(h, wih_ref[...], (((1,), (1,)), ((), ())),
                             preferred_element_type=jnp.float32) + bih_ref[...]
    gh = jax.lax.dot_general(hp, whh_ref[...], (((1,), (1,)), ((), ())),
                             preferred_element_type=jnp.float32) + bhh_ref[...]
    r = jax.nn.sigmoid(gi[:, :d] + gh[:, :d])
    z = jax.nn.sigmoid(gi[:, d:2 * d] + gh[:, d:2 * d])
    n = jnp.tanh(gi[:, 2 * d:] + r * gh[:, 2 * d:])
    g = (1.0 - z) * n + z * hp
    o_ref[...] = g * jax.nn.sigmoid(g)


def _lin_kernel(out_ref, wl_ref, bl_ref, o_ref):
    o_ref[...] = jax.lax.dot_general(out_ref[...], wl_ref[...],
                                     (((1,), (1,)), ((), ())),
                                     preferred_element_type=jnp.float32) + bl_ref[...]


def kernel(x, edge_index, batch, W_src, W_dst, att_src, att_dst, bias_gat,
           W_ih, W_hh, b_ih, b_hh, W_lin, b_lin):
    n, d = x.shape
    b = _B
    out_dim = W_lin.shape[0]
    grid = (n // _NB,)
    batch2 = batch.reshape(n, 1)

    xs, a_src, out, m_src = pl.pallas_call(
        _pre_kernel,
        grid=grid,
        in_specs=[
            pl.BlockSpec((_NB, d), lambda i: (i, 0)),
            pl.BlockSpec((_NB, 1), lambda i: (i, 0)),
            pl.BlockSpec((d, d), lambda i: (0, 0)),
            pl.BlockSpec((d, 1), lambda i: (0, 0)),
        ],
        out_specs=[
            pl.BlockSpec((_NB, d), lambda i: (i, 0)),
            pl.BlockSpec((_NB, 1), lambda i: (i, 0)),
            pl.BlockSpec((b, d), lambda i: (0, 0)),
            pl.BlockSpec((1, 1), lambda i: (0, 0)),
        ],
        out_shape=[
            jax.ShapeDtypeStruct((n, d), jnp.float32),
            jax.ShapeDtypeStruct((n, 1), jnp.float32),
            jax.ShapeDtypeStruct((b, d), jnp.float32),
            jax.ShapeDtypeStruct((1, 1), jnp.float32),
        ],
    )(x, batch2, W_src, att_src.reshape(d, 1))

    adst_call = pl.pallas_call(
        _adst_kernel,
        out_shape=[
            jax.ShapeDtypeStruct((b, 1), jnp.float32),
            jax.ShapeDtypeStruct((1, 1), jnp.float32),
        ],
    )

    pass_call = pl.pallas_call(
        _pass_kernel,
        grid=grid,
        in_specs=[
            pl.BlockSpec((_NB, d), lambda i: (i, 0)),
            pl.BlockSpec((_NB, 1), lambda i: (i, 0)),
            pl.BlockSpec((_NB, 1), lambda i: (i, 0)),
            pl.BlockSpec((b, 1), lambda i: (0, 0)),
            pl.BlockSpec((1, 1), lambda i: (0, 0)),
        ],
        out_specs=[
            pl.BlockSpec((b, 1), lambda i: (0, 0)),
            pl.BlockSpec((b, d), lambda i: (0, 0)),
        ],
        out_shape=[
            jax.ShapeDtypeStruct((b, 1), jnp.float32),
            jax.ShapeDtypeStruct((b, d), jnp.float32),
        ],
    )

    gru_call = pl.pallas_call(
        _gru_kernel,
        out_shape=jax.ShapeDtypeStruct((b, d), jnp.float32),
    )

    for _ in range(_T):
        a_dst, m = adst_call(out, W_dst, att_dst.reshape(d, 1), m_src)
        den, num = pass_call(xs, a_src, batch2, a_dst, m)
        out = gru_call(den, num, bias_gat.reshape(1, d), out, W_ih, W_hh,
                       b_ih.reshape(1, 3 * d), b_hh.reshape(1, 3 * d))

    return pl.pallas_call(
        _lin_kernel,
        out_shape=jax.ShapeDtypeStruct((b, out_dim), jnp.float32),
    )(out, W_lin, b_lin.reshape(1, out_dim))
